# baseline (speedup 1.0000x reference)
import functools

import jax
import jax.numpy as jnp
from jax import lax
from jax.experimental import pallas as pl
from jax.experimental.pallas import tpu as pltpu
from jax.experimental.pallas import tpu_sc as plsc

N = 8192
L = 64
P = L * L

_INFO = plsc.get_sparse_core_info()
_NC, _NS = _INFO.num_cores, _INFO.num_subcores
NW = _NC * _NS
ROWS_PER_W = N // NW
ZROWS = 16
NZCOPY = ROWS_PER_W // ZROWS
NVEC = ROWS_PER_W // 16


@functools.partial(
    pl.kernel,
    mesh=plsc.VectorSubcoreMesh(core_axis_name="c", subcore_axis_name="s"),
    out_type=jax.ShapeDtypeStruct((N * P,), jnp.float32),
    scratch_types=[
        pltpu.VMEM((ROWS_PER_W,), jnp.int32),
        pltpu.VMEM((ROWS_PER_W,), jnp.int32),
        pltpu.VMEM((ZROWS * P,), jnp.float32),
        pltpu.VMEM((4, 128), jnp.int32),
        pltpu.VMEM((4, 128), jnp.float32),
        pltpu.SemaphoreType.DMA,
    ],
)
def _sc_masks(y_hbm, x_hbm, zero_hbm, out_hbm, y_v, x_v, zbuf, idx_v, val_v, sem):
    wid = lax.axis_index("s") * _NC + lax.axis_index("c")
    base_row = wid * ROWS_PER_W
    pltpu.sync_copy(y_hbm.at[pl.ds(base_row, ROWS_PER_W)], y_v)
    pltpu.sync_copy(x_hbm.at[pl.ds(base_row, ROWS_PER_W)], x_v)
    pltpu.sync_copy(zero_hbm, zbuf)

    copies = []
    for cc in range(NZCOPY):
        dst = out_hbm.at[pl.ds((base_row + cc * ZROWS) * P, ZROWS * P)]
        copies.append(pltpu.async_copy(zbuf, dst, sem))

    iota16 = lax.iota(jnp.int32, 16)
    neg1 = jnp.full((16,), -1.0, jnp.float32)
    for cc in range(NVEC):
        yv = y_v[pl.ds(cc * 16, 16)]
        xv = x_v[pl.ds(cc * 16, 16)]
        sep = yv * L + xv
        org_idx = (base_row + cc * 16 + iota16) * P
        sep_idx = org_idx + sep
        org_val = jnp.where(sep == 0, -1.0, 1.0).astype(jnp.float32)
        j, k = divmod(cc, 8)
        idx_v[j, pl.ds(k * 16, 16)] = org_idx
        val_v[j, pl.ds(k * 16, 16)] = org_val
        idx_v[2 + j, pl.ds(k * 16, 16)] = sep_idx
        val_v[2 + j, pl.ds(k * 16, 16)] = neg1

    for c in copies:
        c.wait()
    for j in range(4):
        pltpu.sync_copy(val_v.at[j], out_hbm.at[idx_v.at[j]])


def kernel(lats, x_seps, y_seps):
    y = y_seps.astype(jnp.int32)
    x = x_seps.astype(jnp.int32)
    zero_tpl = jnp.zeros((ZROWS * P,), jnp.float32)
    flat = _sc_masks(y, x, zero_tpl)
    return flat.reshape(N, 1, L, L)
